# baseline (speedup 1.0000x reference)
import functools

import jax
import jax.numpy as jnp
from jax import lax
from jax.experimental import pallas as pl
from jax.experimental.pallas import tpu as pltpu
from jax.experimental.pallas import tpu_sc as plsc

T = 2048
D = 768
E = 8
F = 1536
L = 128
TM = 128
NT = 24
PT = NT * TM
CB = 256
NW = 32
CH = T // NW
CS = 4
CT = NT // CS


def _router_body(x_ref, rw_ref, dest_ref, w_ref, lb_ref,
                 rexp_ref, rstart_ref, rlen_ref, nruns_ref):
    x = x_ref[...]
    rw = rw_ref[...]
    logits = lax.dot_general(x, rw, (((1,), (1,)), ((), ())),
                             preferred_element_type=jnp.float32)
    lane = lax.broadcasted_iota(jnp.int32, (T, L), 1)
    logits = jnp.where(lane < E, logits, jnp.float32(-3.0e38))
    m = jnp.max(logits, axis=1, keepdims=True)
    p_un = jnp.exp(logits - m)
    s = jnp.sum(p_un, axis=1, keepdims=True)
    probs = p_un / s
    pmax = jnp.max(probs, axis=1, keepdims=True)
    eid = jnp.min(jnp.where(probs >= pmax, lane, L), axis=1, keepdims=True)
    onehot = (lane == eid).astype(jnp.float32)
    row = lax.broadcasted_iota(jnp.int32, (CB, CB), 0)
    col = lax.broadcasted_iota(jnp.int32, (CB, CB), 1)
    tri = (col < row).astype(jnp.float32)
    running = jnp.zeros((1, L), jnp.float32)
    ranks = []
    for j in range(T // CB):
        blk = lax.slice(onehot, (j * CB, 0), ((j + 1) * CB, L))
        within = lax.dot_general(tri, blk, (((1,), (0,)), ((), ())),
                                 preferred_element_type=jnp.float32)
        ranks.append(within + running)
        running = running + jnp.sum(blk, axis=0, keepdims=True)
    ranks_full = jnp.concatenate(ranks, axis=0)
    rank_t = jnp.sum(ranks_full * onehot, axis=1, keepdims=True)
    ntiles = jnp.floor((running + 127.0) * (1.0 / 128.0))
    rowl = lax.broadcasted_iota(jnp.int32, (L, L), 0)
    coll = lax.broadcasted_iota(jnp.int32, (L, L), 1)
    tril = (coll < rowl).astype(jnp.float32)
    aoff = lax.dot_general(ntiles, tril, (((1,), (1,)), ((), ())),
                           preferred_element_type=jnp.float32) * 128.0
    aoff_t = jnp.sum(aoff * onehot, axis=1, keepdims=True)
    dest_ref[...] = (aoff_t + rank_t).astype(jnp.int32)
    w = pmax / (pmax + 1e-6)
    w_ref[...] = jnp.broadcast_to(w, (T, L))
    psum = jnp.sum(probs, axis=0, keepdims=True)
    total = jnp.sum(running)
    frac = running / (total + 1e-6)
    lb_ref[...] = jnp.reshape(jnp.sum(frac * psum) * E, (1, 1))
    tstart = aoff * (1.0 / 128.0)
    act = (running > 0.0).astype(jnp.float32)
    cumincl = lax.dot_general(act, (coll <= rowl).astype(jnp.float32),
                              (((1,), (1,)), ((), ())),
                              preferred_element_type=jnp.float32)
    rowj = lax.broadcasted_iota(jnp.int32, (MAXR, L), 0).astype(jnp.float32)
    lanef = lax.broadcasted_iota(jnp.int32, (MAXR, L), 1).astype(jnp.float32)
    mself = ((jnp.broadcast_to(cumincl, (MAXR, L)) == rowj + 1.0)
             .astype(jnp.float32) * jnp.broadcast_to(act, (MAXR, L)))
    rexp_ref[...] = jnp.sum(mself * lanef, axis=1, keepdims=True).astype(jnp.int32)
    rstart_ref[...] = jnp.sum(mself * tstart, axis=1, keepdims=True).astype(jnp.int32)
    rlen_ref[...] = jnp.sum(mself * ntiles, axis=1, keepdims=True).astype(jnp.int32)
    nruns_ref[...] = jnp.reshape(jnp.sum(act), (1, 1)).astype(jnp.int32)


def _router(xf, rw_pad):
    return pl.pallas_call(
        _router_body,
        out_shape=[
            jax.ShapeDtypeStruct((T, 1), jnp.int32),
            jax.ShapeDtypeStruct((T, L), jnp.float32),
            jax.ShapeDtypeStruct((1, 1), jnp.float32),
            jax.ShapeDtypeStruct((MAXR, 1), jnp.int32),
            jax.ShapeDtypeStruct((MAXR, 1), jnp.int32),
            jax.ShapeDtypeStruct((MAXR, 1), jnp.int32),
            jax.ShapeDtypeStruct((1, 1), jnp.int32),
        ],
    )(xf, rw_pad)


def _dispatch(xf, w16, dest):
    mesh = plsc.VectorSubcoreMesh(core_axis_name="c", subcore_axis_name="s")

    @functools.partial(
        pl.kernel,
        out_type=[jax.ShapeDtypeStruct((PT, D), jnp.float32),
                  jax.ShapeDtypeStruct((PT, L), jnp.float32)],
        mesh=mesh,
        scratch_types=[pltpu.VMEM((CH,), jnp.int32),
                       pltpu.VMEM((CH, D), jnp.float32),
                       pltpu.VMEM((CH, L), jnp.float32),
                       pltpu.SemaphoreType.DMA,
                       pltpu.SemaphoreType.DMA,
                       pltpu.SemaphoreType.DMA],
    )
    def k(x_hbm, w_hbm, dest_hbm, xs_hbm, ws_hbm,
          dest_v, rows_v, wrows_v, sem1, sem2, sem3):
        wid = lax.axis_index("s") * 2 + lax.axis_index("c")
        base = wid * CH
        i0 = pltpu.async_copy(dest_hbm.at[pl.ds(base, CH)], dest_v, sem1)
        i1 = pltpu.async_copy(x_hbm.at[pl.ds(base, CH)], rows_v, sem2)
        i2 = pltpu.async_copy(w_hbm.at[pl.ds(base, CH)], wrows_v, sem3)
        i0.wait()
        i1.wait()
        i2.wait()
        c1 = pltpu.async_copy(rows_v, xs_hbm.at[dest_v], sem1)
        c2 = pltpu.async_copy(wrows_v, ws_hbm.at[dest_v], sem2)
        c1.wait()
        c2.wait()

    return k(xf, w16, dest)


MAXR = E


def _ffn2_body(rexp_s, rstart_s, rlen_s, nruns_s,
               xs_hbm, ws_ref, f1_hbm, g1_hbm, f2_hbm,
               b1_ref, bg_ref, b2_ref, out_hbm,
               w1_v, wg_v, w2_v, xs_v, st_v, sems, semx, semo):
    def issue(j, slot):
        e = rexp_s[j, 0]
        pltpu.make_async_copy(f1_hbm.at[pl.ds(e, 1)], w1_v.at[pl.ds(slot, 1)],
                              sems.at[0, slot]).start()
        pltpu.make_async_copy(g1_hbm.at[pl.ds(e, 1)], wg_v.at[pl.ds(slot, 1)],
                              sems.at[1, slot]).start()
        pltpu.make_async_copy(f2_hbm.at[pl.ds(e, 1)], w2_v.at[pl.ds(slot, 1)],
                              sems.at[2, slot]).start()

    def wait(j, slot):
        e = rexp_s[j, 0]
        pltpu.make_async_copy(f1_hbm.at[pl.ds(e, 1)], w1_v.at[pl.ds(slot, 1)],
                              sems.at[0, slot]).wait()
        pltpu.make_async_copy(g1_hbm.at[pl.ds(e, 1)], wg_v.at[pl.ds(slot, 1)],
                              sems.at[1, slot]).wait()
        pltpu.make_async_copy(f2_hbm.at[pl.ds(e, 1)], w2_v.at[pl.ds(slot, 1)],
                              sems.at[2, slot]).wait()

    def xs_chunk(c):
        return pltpu.make_async_copy(
            xs_hbm.at[pl.ds(c * CS * TM, CS * TM)],
            xs_v.at[pl.ds(c * CS * TM, CS * TM)], semx.at[c])

    def out_copy(gslot, gt):
        return pltpu.make_async_copy(
            st_v.at[pl.ds(gslot * TM, TM)],
            out_hbm.at[pl.ds(gt * TM, TM)], semo.at[gslot])

    nruns = nruns_s[0, 0]

    @pl.when(0 < nruns)
    def _():
        issue(0, 0)
    for c in range(CT):
        xs_chunk(c).start()

    @pl.when(1 < nruns)
    def _():
        issue(1, 1)

    for j in range(MAXR):
        slot = j % 2

        @pl.when(j < nruns)
        def _():
            rstart = rstart_s[j, 0]
            rlen = rlen_s[j, 0]
            chunk_lo = (rstart + CS - 1) // CS
            chunk_hi = (rstart + rlen + CS - 1) // CS
            for c in range(CT):
                @pl.when((c >= chunk_lo) & (c < chunk_hi))
                def _():
                    xs_chunk(c).wait()
            wait(j, slot)
            e = rexp_s[j, 0]
            b1 = b1_ref[pl.ds(e, 1), 0, :]
            bg = bg_ref[pl.ds(e, 1), 0, :]
            b2 = b2_ref[pl.ds(e, 1), 0, :]

            def ffn_rows(base, n):
                x = xs_v[pl.ds(base, n * TM), :]
                h = lax.dot_general(x, w1_v[slot], (((1,), (1,)), ((), ())),
                                    preferred_element_type=jnp.float32) + b1
                g = lax.dot_general(x, wg_v[slot], (((1,), (1,)), ((), ())),
                                    preferred_element_type=jnp.float32) + bg
                a = g * lax.logistic(g) * h
                y = lax.dot_general(a, w2_v[slot], (((1,), (1,)), ((), ())),
                                    preferred_element_type=jnp.float32) + b2
                return y * ws_ref[pl.ds(base, n * TM), 0:1]

            def emit_tile(gt, yt):
                gslot = lax.rem(gt, 2)

                @pl.when(gt >= 2)
                def _():
                    out_copy(gslot, gt - 2).wait()
                st_v[pl.ds(gslot * TM, TM), :] = yt
                out_copy(gslot, gt).start()

            def quad(t, carry):
                gt = rstart + 4 * t
                y4 = ffn_rows(gt * TM, 4)
                for u in range(4):
                    emit_tile(gt + u, y4[u * TM:(u + 1) * TM])
                return carry

            lax.fori_loop(0, rlen // 4, quad, jnp.int32(0))
            rem = lax.rem(rlen, 4)

            @pl.when(rem >= 2)
            def _():
                gt = rstart + rlen - rem
                y2 = ffn_rows(gt * TM, 2)
                emit_tile(gt, y2[:TM])
                emit_tile(gt + 1, y2[TM:])

            @pl.when(lax.rem(rem, 2) == 1)
            def _():
                gt = rstart + rlen - 1
                emit_tile(gt, ffn_rows(gt * TM, 1))

        if j + 2 < MAXR:
            @pl.when(j + 2 < nruns)
            def _():
                issue(j + 2, slot)

    last = nruns - 1
    total = rstart_s[last, 0] + rlen_s[last, 0]
    for q in range(2):
        out_copy(lax.rem(total - 2 + q, 2), total - 2 + q).wait()
    total_chunks = (total + CS - 1) // CS
    for c in range(CT):
        @pl.when(c >= total_chunks)
        def _():
            xs_chunk(c).wait()


def _ffn2(rexp, rstart, rlen, nruns, xs, ws, fc1_w, gate_w, fc2_w, b1, bg, b2):
    smem = pl.BlockSpec(memory_space=pltpu.SMEM)
    anyspace = pl.BlockSpec(memory_space=pl.ANY)
    return pl.pallas_call(
        _ffn2_body,
        in_specs=[smem, smem, smem, smem,
                  anyspace,
                  pl.BlockSpec(memory_space=pltpu.VMEM),
                  anyspace, anyspace, anyspace,
                  pl.BlockSpec(memory_space=pltpu.VMEM),
                  pl.BlockSpec(memory_space=pltpu.VMEM),
                  pl.BlockSpec(memory_space=pltpu.VMEM)],
        out_specs=anyspace,
        out_shape=jax.ShapeDtypeStruct((PT, D), jnp.float32),
        scratch_shapes=[pltpu.VMEM((2, F, D), jnp.float32),
                        pltpu.VMEM((2, F, D), jnp.float32),
                        pltpu.VMEM((2, D, F), jnp.float32),
                        pltpu.VMEM((PT, D), jnp.float32),
                        pltpu.VMEM((2 * TM, D), jnp.float32),
                        pltpu.SemaphoreType.DMA((3, 2)),
                        pltpu.SemaphoreType.DMA((CT,)),
                        pltpu.SemaphoreType.DMA((2,))],
    )(rexp, rstart, rlen, nruns, xs, ws, fc1_w, gate_w, fc2_w, b1, bg, b2)


def _combine(ys, dest):
    mesh = plsc.VectorSubcoreMesh(core_axis_name="c", subcore_axis_name="s")

    @functools.partial(
        pl.kernel,
        out_type=jax.ShapeDtypeStruct((T, D), jnp.float32),
        mesh=mesh,
        scratch_types=[pltpu.VMEM((CH,), jnp.int32),
                       pltpu.VMEM((CH, D), jnp.float32),
                       pltpu.SemaphoreType.DMA],
    )
    def k(ys_hbm, dest_hbm, out_hbm, dest_v, rows_v, sem):
        wid = lax.axis_index("s") * 2 + lax.axis_index("c")
        base = wid * CH
        pltpu.sync_copy(dest_hbm.at[pl.ds(base, CH)], dest_v)
        pltpu.async_copy(ys_hbm.at[dest_v], rows_v, sem).wait()
        pltpu.sync_copy(rows_v, out_hbm.at[pl.ds(base, CH)])

    return k(ys, dest)


def kernel(x, router_w, fc1_w, fc1_b, gate_w, gate_b, fc2_w, fc2_b):
    Bq, Nq, C = x.shape
    xf = x.reshape(T, D)
    rw_pad = jnp.zeros((L, D), jnp.float32).at[:E].set(router_w)
    dest2, w16, lb2, rexp, rstart, rlen, nruns = _router(xf, rw_pad)
    dest = dest2.reshape(T)
    xs, ws = _dispatch(xf, w16, dest)
    ys = _ffn2(rexp, rstart, rlen, nruns, xs, ws, fc1_w, gate_w, fc2_w,
               fc1_b.reshape(E, 1, F), gate_b.reshape(E, 1, F),
               fc2_b.reshape(E, 1, D))
    out = _combine(ys, dest)
    return (out.reshape(Bq, Nq, C), lb2[0, 0])


# speedup vs baseline: 1.1322x; 1.1322x over previous
import functools

import jax
import jax.numpy as jnp
from jax import lax
from jax.experimental import pallas as pl
from jax.experimental.pallas import tpu as pltpu
from jax.experimental.pallas import tpu_sc as plsc

T = 2048
D = 768
E = 8
F = 1536
L = 128
TM = 128
NT = 24
PT = NT * TM
CB = 256
NW = 32
CH = T // NW
CS = 4
CT = NT // CS


def _router_body(x_ref, rw_ref, dest_ref, w_ref, lb_ref,
                 rexp_ref, rstart_ref, rlen_ref, nruns_ref):
    x = x_ref[...]
    rw = rw_ref[...]
    logits = lax.dot_general(x, rw, (((1,), (1,)), ((), ())),
                             preferred_element_type=jnp.float32)
    lane = lax.broadcasted_iota(jnp.int32, (T, L), 1)
    logits = jnp.where(lane < E, logits, jnp.float32(-3.0e38))
    m = jnp.max(logits, axis=1, keepdims=True)
    p_un = jnp.exp(logits - m)
    s = jnp.sum(p_un, axis=1, keepdims=True)
    probs = p_un / s
    pmax = jnp.max(probs, axis=1, keepdims=True)
    eid = jnp.min(jnp.where(probs >= pmax, lane, L), axis=1, keepdims=True)
    onehot = (lane == eid).astype(jnp.float32)
    row = lax.broadcasted_iota(jnp.int32, (CB, CB), 0)
    col = lax.broadcasted_iota(jnp.int32, (CB, CB), 1)
    tri = (col < row).astype(jnp.float32)
    running = jnp.zeros((1, L), jnp.float32)
    ranks = []
    for j in range(T // CB):
        blk = lax.slice(onehot, (j * CB, 0), ((j + 1) * CB, L))
        within = lax.dot_general(tri, blk, (((1,), (0,)), ((), ())),
                                 preferred_element_type=jnp.float32)
        ranks.append(within + running)
        running = running + jnp.sum(blk, axis=0, keepdims=True)
    ranks_full = jnp.concatenate(ranks, axis=0)
    rank_t = jnp.sum(ranks_full * onehot, axis=1, keepdims=True)
    ntiles = jnp.floor((running + 127.0) * (1.0 / 128.0))
    rowl = lax.broadcasted_iota(jnp.int32, (L, L), 0)
    coll = lax.broadcasted_iota(jnp.int32, (L, L), 1)
    tril = (coll < rowl).astype(jnp.float32)
    aoff = lax.dot_general(ntiles, tril, (((1,), (1,)), ((), ())),
                           preferred_element_type=jnp.float32) * 128.0
    aoff_t = jnp.sum(aoff * onehot, axis=1, keepdims=True)
    dest_ref[...] = (aoff_t + rank_t).astype(jnp.int32)
    w = pmax / (pmax + 1e-6)
    w_ref[...] = jnp.broadcast_to(w, (T, L))
    psum = jnp.sum(probs, axis=0, keepdims=True)
    total = jnp.sum(running)
    frac = running / (total + 1e-6)
    lb_ref[...] = jnp.reshape(jnp.sum(frac * psum) * E, (1, 1))
    tstart = aoff * (1.0 / 128.0)
    act = (running > 0.0).astype(jnp.float32)
    cumincl = lax.dot_general(act, (coll <= rowl).astype(jnp.float32),
                              (((1,), (1,)), ((), ())),
                              preferred_element_type=jnp.float32)
    rowj = lax.broadcasted_iota(jnp.int32, (MAXR, L), 0).astype(jnp.float32)
    lanef = lax.broadcasted_iota(jnp.int32, (MAXR, L), 1).astype(jnp.float32)
    mself = ((jnp.broadcast_to(cumincl, (MAXR, L)) == rowj + 1.0)
             .astype(jnp.float32) * jnp.broadcast_to(act, (MAXR, L)))
    rexp_ref[...] = jnp.sum(mself * lanef, axis=1, keepdims=True).astype(jnp.int32)
    rstart_ref[...] = jnp.sum(mself * tstart, axis=1, keepdims=True).astype(jnp.int32)
    rlen_ref[...] = jnp.sum(mself * ntiles, axis=1, keepdims=True).astype(jnp.int32)
    nruns_ref[...] = jnp.reshape(jnp.sum(act), (1, 1)).astype(jnp.int32)


def _router(xf, rw_pad):
    return pl.pallas_call(
        _router_body,
        out_shape=[
            jax.ShapeDtypeStruct((T, 1), jnp.int32),
            jax.ShapeDtypeStruct((T, L), jnp.float32),
            jax.ShapeDtypeStruct((1, 1), jnp.float32),
            jax.ShapeDtypeStruct((MAXR, 1), jnp.int32),
            jax.ShapeDtypeStruct((MAXR, 1), jnp.int32),
            jax.ShapeDtypeStruct((MAXR, 1), jnp.int32),
            jax.ShapeDtypeStruct((1, 1), jnp.int32),
        ],
    )(xf, rw_pad)


def _dispatch(xf, w16, dest):
    mesh = plsc.VectorSubcoreMesh(core_axis_name="c", subcore_axis_name="s")

    @functools.partial(
        pl.kernel,
        out_type=[jax.ShapeDtypeStruct((PT, D), jnp.float32),
                  jax.ShapeDtypeStruct((PT, L), jnp.float32)],
        mesh=mesh,
        scratch_types=[pltpu.VMEM((CH,), jnp.int32),
                       pltpu.VMEM((CH, D), jnp.float32),
                       pltpu.VMEM((CH, L), jnp.float32),
                       pltpu.SemaphoreType.DMA,
                       pltpu.SemaphoreType.DMA,
                       pltpu.SemaphoreType.DMA],
    )
    def k(x_hbm, w_hbm, dest_hbm, xs_hbm, ws_hbm,
          dest_v, rows_v, wrows_v, sem1, sem2, sem3):
        wid = lax.axis_index("s") * 2 + lax.axis_index("c")
        base = wid * CH
        i0 = pltpu.async_copy(dest_hbm.at[pl.ds(base, CH)], dest_v, sem1)
        i1 = pltpu.async_copy(x_hbm.at[pl.ds(base, CH)], rows_v, sem2)
        i2 = pltpu.async_copy(w_hbm.at[pl.ds(base, CH)], wrows_v, sem3)
        i0.wait()
        i1.wait()
        i2.wait()
        c1 = pltpu.async_copy(rows_v, xs_hbm.at[dest_v], sem1)
        c2 = pltpu.async_copy(wrows_v, ws_hbm.at[dest_v], sem2)
        c1.wait()
        c2.wait()

    return k(xf, w16, dest)


MAXR = E


def _ffn2_body(rexp_s, rstart_s, rlen_s, nruns_s,
               xs_hbm, ws_ref, f1_hbm, g1_hbm, f2_hbm,
               b1_ref, bg_ref, b2_ref, out_hbm,
               w1_v, wg_v, w2_v, xs_v, st_v, sems, semx, semo):
    def issue(j, slot):
        e = rexp_s[j, 0]
        pltpu.make_async_copy(f1_hbm.at[pl.ds(e, 1)], w1_v.at[pl.ds(slot, 1)],
                              sems.at[0, slot]).start()
        pltpu.make_async_copy(g1_hbm.at[pl.ds(e, 1)], wg_v.at[pl.ds(slot, 1)],
                              sems.at[1, slot]).start()
        pltpu.make_async_copy(f2_hbm.at[pl.ds(e, 1)], w2_v.at[pl.ds(slot, 1)],
                              sems.at[2, slot]).start()

    def wait(j, slot):
        e = rexp_s[j, 0]
        pltpu.make_async_copy(f1_hbm.at[pl.ds(e, 1)], w1_v.at[pl.ds(slot, 1)],
                              sems.at[0, slot]).wait()
        pltpu.make_async_copy(g1_hbm.at[pl.ds(e, 1)], wg_v.at[pl.ds(slot, 1)],
                              sems.at[1, slot]).wait()
        pltpu.make_async_copy(f2_hbm.at[pl.ds(e, 1)], w2_v.at[pl.ds(slot, 1)],
                              sems.at[2, slot]).wait()

    def xs_chunk(c):
        return pltpu.make_async_copy(
            xs_hbm.at[pl.ds(c * CS * TM, CS * TM)],
            xs_v.at[pl.ds(c * CS * TM, CS * TM)], semx.at[c])

    def out_copy(gslot, gt):
        return pltpu.make_async_copy(
            st_v.at[pl.ds(gslot * TM, TM)],
            out_hbm.at[pl.ds(gt * TM, TM)], semo.at[gslot])

    nruns = nruns_s[0, 0]

    @pl.when(0 < nruns)
    def _():
        issue(0, 0)
    for c in range(CT):
        xs_chunk(c).start()

    @pl.when(1 < nruns)
    def _():
        issue(1, 1)

    for j in range(MAXR):
        slot = j % 3

        @pl.when(j < nruns)
        def _():
            rstart = rstart_s[j, 0]
            rlen = rlen_s[j, 0]
            chunk_lo = (rstart + CS - 1) // CS
            chunk_hi = (rstart + rlen + CS - 1) // CS
            for c in range(CT):
                @pl.when((c >= chunk_lo) & (c < chunk_hi))
                def _():
                    xs_chunk(c).wait()
            wait(j, slot)
            if j + 2 < MAXR:
                @pl.when(j + 2 < nruns)
                def _():
                    issue(j + 2, (j + 2) % 3)
            e = rexp_s[j, 0]
            b1 = b1_ref[pl.ds(e, 1), 0, :]
            bg = bg_ref[pl.ds(e, 1), 0, :]
            b2 = b2_ref[pl.ds(e, 1), 0, :]

            def ffn_rows(base, n):
                x = xs_v[pl.ds(base, n * TM), :]
                h = lax.dot_general(x, w1_v[slot], (((1,), (1,)), ((), ())),
                                    preferred_element_type=jnp.float32) + b1
                g = lax.dot_general(x, wg_v[slot], (((1,), (1,)), ((), ())),
                                    preferred_element_type=jnp.float32) + bg
                a = g * lax.logistic(g) * h
                y = lax.dot_general(a, w2_v[slot], (((1,), (1,)), ((), ())),
                                    preferred_element_type=jnp.float32) + b2
                return y * ws_ref[pl.ds(base, n * TM), 0:1]

            def emit_tile(gt, yt):
                gslot = lax.rem(gt, 2)

                @pl.when(gt >= 2)
                def _():
                    out_copy(gslot, gt - 2).wait()
                st_v[pl.ds(gslot * TM, TM), :] = yt
                out_copy(gslot, gt).start()

            def pair(t, carry):
                gt = rstart + 2 * t
                y2 = ffn_rows(gt * TM, 2)
                emit_tile(gt, y2[:TM])
                emit_tile(gt + 1, y2[TM:])
                return carry

            lax.fori_loop(0, rlen // 2, pair, jnp.int32(0))

            @pl.when(lax.rem(rlen, 2) == 1)
            def _():
                gt = rstart + rlen - 1
                emit_tile(gt, ffn_rows(gt * TM, 1))


    last = nruns - 1
    total = rstart_s[last, 0] + rlen_s[last, 0]
    for q in range(2):
        out_copy(lax.rem(total - 2 + q, 2), total - 2 + q).wait()
    total_chunks = (total + CS - 1) // CS
    for c in range(CT):
        @pl.when(c >= total_chunks)
        def _():
            xs_chunk(c).wait()


def _ffn2(rexp, rstart, rlen, nruns, xs, ws, fc1_w, gate_w, fc2_w, b1, bg, b2):
    smem = pl.BlockSpec(memory_space=pltpu.SMEM)
    anyspace = pl.BlockSpec(memory_space=pl.ANY)
    return pl.pallas_call(
        _ffn2_body,
        in_specs=[smem, smem, smem, smem,
                  anyspace,
                  pl.BlockSpec(memory_space=pltpu.VMEM),
                  anyspace, anyspace, anyspace,
                  pl.BlockSpec(memory_space=pltpu.VMEM),
                  pl.BlockSpec(memory_space=pltpu.VMEM),
                  pl.BlockSpec(memory_space=pltpu.VMEM)],
        out_specs=anyspace,
        out_shape=jax.ShapeDtypeStruct((PT, D), jnp.float32),
        compiler_params=pltpu.CompilerParams(
            vmem_limit_bytes=100 * 1024 * 1024),
        scratch_shapes=[pltpu.VMEM((3, F, D), jnp.float32),
                        pltpu.VMEM((3, F, D), jnp.float32),
                        pltpu.VMEM((3, D, F), jnp.float32),
                        pltpu.VMEM((PT, D), jnp.float32),
                        pltpu.VMEM((2 * TM, D), jnp.float32),
                        pltpu.SemaphoreType.DMA((3, 3)),
                        pltpu.SemaphoreType.DMA((CT,)),
                        pltpu.SemaphoreType.DMA((2,))],
    )(rexp, rstart, rlen, nruns, xs, ws, fc1_w, gate_w, fc2_w, b1, bg, b2)


def _combine(ys, dest):
    mesh = plsc.VectorSubcoreMesh(core_axis_name="c", subcore_axis_name="s")

    @functools.partial(
        pl.kernel,
        out_type=jax.ShapeDtypeStruct((T, D), jnp.float32),
        mesh=mesh,
        scratch_types=[pltpu.VMEM((CH,), jnp.int32),
                       pltpu.VMEM((CH, D), jnp.float32),
                       pltpu.SemaphoreType.DMA],
    )
    def k(ys_hbm, dest_hbm, out_hbm, dest_v, rows_v, sem):
        wid = lax.axis_index("s") * 2 + lax.axis_index("c")
        base = wid * CH
        pltpu.sync_copy(dest_hbm.at[pl.ds(base, CH)], dest_v)
        pltpu.async_copy(ys_hbm.at[dest_v], rows_v, sem).wait()
        pltpu.sync_copy(rows_v, out_hbm.at[pl.ds(base, CH)])

    return k(ys, dest)


def kernel(x, router_w, fc1_w, fc1_b, gate_w, gate_b, fc2_w, fc2_b):
    Bq, Nq, C = x.shape
    xf = x.reshape(T, D)
    rw_pad = jnp.zeros((L, D), jnp.float32).at[:E].set(router_w)
    dest2, w16, lb2, rexp, rstart, rlen, nruns = _router(xf, rw_pad)
    dest = dest2.reshape(T)
    xs, ws = _dispatch(xf, w16, dest)
    ys = _ffn2(rexp, rstart, rlen, nruns, xs, ws, fc1_w, gate_w, fc2_w,
               fc1_b.reshape(E, 1, F), gate_b.reshape(E, 1, F),
               fc2_b.reshape(E, 1, D))
    out = _combine(ys, dest)
    return (out.reshape(Bq, Nq, C), lb2[0, 0])
